# Initial kernel scaffold; baseline (speedup 1.0000x reference)
import functools

import jax
import jax.numpy as jnp
from jax import lax
from jax.experimental import pallas as pl
from jax.experimental.pallas import tpu as pltpu
from jax.experimental.pallas import tpu_sc as plsc

N_SEG = 10000
N_INC = 320000
D = 128
CW = 16
NC = 2
NS = 16
NW = NC * NS
PER_TILE = N_INC // NW
B = 80
STEPS = PER_TILE // B
SEG_PER_TILE = N_SEG // NS
ZR = 125

_mesh = plsc.VectorSubcoreMesh(
    core_axis_name="c", subcore_axis_name="s", num_cores=NC, num_subcores=NS
)


@functools.partial(
    pl.kernel,
    out_type=(
        jax.ShapeDtypeStruct((NC, N_SEG, D), jnp.float32),
        jax.ShapeDtypeStruct((NC, N_SEG, CW), jnp.float32),
    ),
    mesh=_mesh,
    scratch_types=[
        pltpu.VMEM_SHARED((N_SEG, D), jnp.float32),
        pltpu.VMEM_SHARED((N_SEG, CW), jnp.float32),
        pltpu.VMEM((B, D), jnp.float32),
        pltpu.VMEM((B, D), jnp.float32),
        pltpu.VMEM((B,), jnp.int32),
        pltpu.VMEM((B,), jnp.int32),
        pltpu.VMEM((B, CW), jnp.float32),
        pltpu.VMEM((ZR, D), jnp.float32),
        pltpu.VMEM((ZR, CW), jnp.float32),
        pltpu.SemaphoreType.DMA,
        pltpu.SemaphoreType.DMA,
    ],
)
def _scatter_partials(x_hbm, seg_hbm, psum_hbm, pcnt_hbm,
                      acc, cnt, xb0, xb1, ib0, ib1, ones, zb, zcb,
                      sem0, sem1):
    cid = lax.axis_index("c")
    sid = lax.axis_index("s")
    wid = cid * NS + sid
    base = wid * PER_TILE

    zero16 = jnp.zeros((16,), jnp.float32)
    one16 = jnp.ones((16,), jnp.float32)

    def _zrow(i, _):
        def _zcol(j, _):
            zb[i, pl.ds(j * 16, 16)] = zero16
            return 0
        lax.fori_loop(0, D // 16, _zcol, 0)
        zcb[i, :] = zero16
        return 0

    lax.fori_loop(0, ZR, _zrow, 0)

    def _orow(i, _):
        ones[i, :] = one16
        return 0

    lax.fori_loop(0, B, _orow, 0)

    rbase = sid * SEG_PER_TILE

    def _zchunk(k, _):
        pltpu.sync_copy(zb, acc.at[pl.ds(rbase + k * ZR, ZR)])
        pltpu.sync_copy(zcb, cnt.at[pl.ds(rbase + k * ZR, ZR)])
        return 0

    lax.fori_loop(0, SEG_PER_TILE // ZR, _zchunk, 0)
    plsc.subcore_barrier()

    def _issue(step, xb, ib, sem):
        off = base + step * B
        pltpu.async_copy(x_hbm.at[pl.ds(off, B)], xb, sem)
        pltpu.async_copy(seg_hbm.at[pl.ds(off, B)], ib, sem)

    def _wait(step, xb, ib, sem):
        off = base + step * B
        pltpu.make_async_copy(x_hbm.at[pl.ds(off, B)], xb, sem).wait()
        pltpu.make_async_copy(seg_hbm.at[pl.ds(off, B)], ib, sem).wait()

    def _scatter_add(xb, ib):
        pltpu.sync_copy(xb, acc.at[ib], add=True)
        pltpu.sync_copy(ones, cnt.at[ib], add=True)

    _issue(0, xb0, ib0, sem0)

    def _body(i, _):
        s0 = 2 * i
        _issue(s0 + 1, xb1, ib1, sem1)
        _wait(s0, xb0, ib0, sem0)
        _scatter_add(xb0, ib0)
        _issue(s0 + 2, xb0, ib0, sem0)
        _wait(s0 + 1, xb1, ib1, sem1)
        _scatter_add(xb1, ib1)
        return 0

    lax.fori_loop(0, (STEPS - 1) // 2, _body, 0)
    _wait(STEPS - 1, xb0, ib0, sem0)
    _scatter_add(xb0, ib0)

    plsc.subcore_barrier()

    pltpu.sync_copy(acc.at[pl.ds(rbase, SEG_PER_TILE)],
                    psum_hbm.at[cid, pl.ds(rbase, SEG_PER_TILE)])
    pltpu.sync_copy(cnt.at[pl.ds(rbase, SEG_PER_TILE)],
                    pcnt_hbm.at[cid, pl.ds(rbase, SEG_PER_TILE)])


_RB = 1000


def _combine_body(ps_ref, pc_ref, o_ref):
    s = ps_ref[0] + ps_ref[1]
    c = pc_ref[0] + pc_ref[1]
    o_ref[...] = s / jnp.maximum(c[:, 0:1], 1.0)


_combine = pl.pallas_call(
    _combine_body,
    grid=(N_SEG // _RB,),
    in_specs=[
        pl.BlockSpec((NC, _RB, D), lambda i: (0, i, 0)),
        pl.BlockSpec((NC, _RB, CW), lambda i: (0, i, 0)),
    ],
    out_specs=pl.BlockSpec((_RB, D), lambda i: (i, 0)),
    out_shape=jax.ShapeDtypeStruct((N_SEG, D), jnp.float32),
)


def kernel(x, hyperedge_index, data, name):
    seg = hyperedge_index[1]
    psum, pcnt = _scatter_partials(x, seg)
    return _combine(psum, pcnt)


# speedup vs baseline: 8.7546x; 8.7546x over previous
import functools

import jax
import jax.numpy as jnp
from jax import lax
from jax.experimental import pallas as pl
from jax.experimental.pallas import tpu as pltpu
from jax.experimental.pallas import tpu_sc as plsc

N_SEG = 10000
SEG_PAD = 10240
N_INC = 320000
D = 128
NC = 2
NS = 16
NW = NC * NS
PER_TILE = N_INC // NW
B = 80
STEPS = PER_TILE // B
SEG_PER_TILE = SEG_PAD // NS
ZR = 128

_mesh = plsc.VectorSubcoreMesh(
    core_axis_name="c", subcore_axis_name="s", num_cores=NC, num_subcores=NS
)


@functools.partial(
    pl.kernel,
    out_type=(
        jax.ShapeDtypeStruct((NC, SEG_PAD, D), jnp.float32),
        jax.ShapeDtypeStruct((NW, SEG_PAD), jnp.float32),
    ),
    mesh=_mesh,
    compiler_params=pltpu.CompilerParams(
        use_tc_tiling_on_sc=False, needs_layout_passes=False
    ),
    scratch_types=[
        pltpu.VMEM_SHARED((SEG_PAD, D), jnp.float32),
        pltpu.VMEM((B, D), jnp.float32),
        pltpu.VMEM((B, D), jnp.float32),
        pltpu.VMEM((B,), jnp.int32),
        pltpu.VMEM((B,), jnp.int32),
        pltpu.VMEM((SEG_PAD,), jnp.float32),
        pltpu.VMEM((ZR, D), jnp.float32),
        pltpu.SemaphoreType.DMA,
        pltpu.SemaphoreType.DMA,
    ],
)
def _scatter_partials(x_hbm, seg_hbm, psum_hbm, pcnt_hbm,
                      acc, xb0, xb1, ib0, ib1, hist, zb, sem0, sem1):
    cid = lax.axis_index("c")
    sid = lax.axis_index("s")
    wid = cid * NS + sid
    base = wid * PER_TILE

    zero16 = jnp.zeros((16,), jnp.float32)
    one16 = jnp.ones((16,), jnp.float32)

    def _zrow(i, _):
        def _zcol(j, _):
            zb[i, pl.ds(j * 16, 16)] = zero16
            return 0
        return lax.fori_loop(0, D // 16, _zcol, 0)

    lax.fori_loop(0, ZR, _zrow, 0)

    def _zhist(i, _):
        hist[pl.ds(i * 16, 16)] = zero16
        return 0

    lax.fori_loop(0, SEG_PAD // 16, _zhist, 0)

    rbase = sid * SEG_PER_TILE

    def _zchunk(k, _):
        pltpu.sync_copy(zb, acc.at[pl.ds(rbase + k * ZR, ZR)])
        return 0

    lax.fori_loop(0, SEG_PER_TILE // ZR, _zchunk, 0)
    plsc.subcore_barrier()

    def _issue(step, xb, ib, sem):
        off = base + step * B
        pltpu.async_copy(x_hbm.at[pl.ds(off, B)], xb, sem)
        pltpu.async_copy(seg_hbm.at[pl.ds(off, B)], ib, sem)

    def _wait(step, xb, ib, sem):
        off = base + step * B
        pltpu.make_async_copy(x_hbm.at[pl.ds(off, B)], xb, sem).wait()
        pltpu.make_async_copy(seg_hbm.at[pl.ds(off, B)], ib, sem).wait()

    def _scatter_add(xb, ib):
        for j in range(B // 16):
            idx = ib[pl.ds(j * 16, 16)]
            plsc.addupdate_scatter(hist, (idx,), one16)
        pltpu.sync_copy(xb, acc.at[ib], add=True)

    _issue(0, xb0, ib0, sem0)

    def _body(i, _):
        s0 = 2 * i
        _issue(s0 + 1, xb1, ib1, sem1)
        _wait(s0, xb0, ib0, sem0)
        _scatter_add(xb0, ib0)
        _issue(s0 + 2, xb0, ib0, sem0)
        _wait(s0 + 1, xb1, ib1, sem1)
        _scatter_add(xb1, ib1)
        return 0

    lax.fori_loop(0, (STEPS - 1) // 2, _body, 0)
    _wait(STEPS - 1, xb0, ib0, sem0)
    _scatter_add(xb0, ib0)

    pltpu.sync_copy(hist, pcnt_hbm.at[wid])
    plsc.subcore_barrier()
    pltpu.sync_copy(acc.at[pl.ds(rbase, SEG_PER_TILE)],
                    psum_hbm.at[cid, pl.ds(rbase, SEG_PER_TILE)])


_RB = 1024


def _combine_body(ps_ref, pc_ref, o_ref):
    s = ps_ref[0] + ps_ref[1]
    c = jnp.sum(pc_ref[...], axis=0)
    o_ref[...] = s / jnp.maximum(c, 1.0)[:, None]


_combine = pl.pallas_call(
    _combine_body,
    grid=(SEG_PAD // _RB,),
    in_specs=[
        pl.BlockSpec((NC, _RB, D), lambda i: (0, i, 0)),
        pl.BlockSpec((NW, _RB), lambda i: (0, i)),
    ],
    out_specs=pl.BlockSpec((_RB, D), lambda i: (i, 0)),
    out_shape=jax.ShapeDtypeStruct((SEG_PAD, D), jnp.float32),
)


def kernel(x, hyperedge_index, data, name):
    seg = hyperedge_index[1]
    psum, pcnt = _scatter_partials(x, seg)
    return _combine(psum, pcnt)[:N_SEG]


# speedup vs baseline: 10.2269x; 1.1682x over previous
import functools

import jax
import jax.numpy as jnp
from jax import lax
from jax.experimental import pallas as pl
from jax.experimental.pallas import tpu as pltpu
from jax.experimental.pallas import tpu_sc as plsc

N_SEG = 10000
SEG_PAD = 10240
N_INC = 320000
D = 128
NC = 2
NS = 16
NW = NC * NS
PER_TILE = N_INC // NW
B = 80
STEPS = PER_TILE // B
SEG_PER_TILE = SEG_PAD // NS
NBUF = 3

_mesh = plsc.VectorSubcoreMesh(
    core_axis_name="c", subcore_axis_name="s", num_cores=NC, num_subcores=NS
)


@functools.partial(
    pl.kernel,
    out_type=(
        jax.ShapeDtypeStruct((NC, SEG_PAD, D), jnp.float32),
        jax.ShapeDtypeStruct((NW, SEG_PAD), jnp.float32),
    ),
    mesh=_mesh,
    compiler_params=pltpu.CompilerParams(
        use_tc_tiling_on_sc=False, needs_layout_passes=False
    ),
    scratch_types=[
        pltpu.VMEM_SHARED((SEG_PAD, D), jnp.float32),
        [pltpu.VMEM((B, D), jnp.float32)] * NBUF,
        [pltpu.VMEM((B,), jnp.int32)] * NBUF,
        pltpu.VMEM((SEG_PAD,), jnp.float32),
        [pltpu.SemaphoreType.DMA] * NBUF,
        [pltpu.SemaphoreType.DMA] * NBUF,
        pltpu.SemaphoreType.DMA,
    ],
)
def _scatter_partials(x_hbm, seg_hbm, psum_hbm, pcnt_hbm,
                      acc, xbs, ibs, hist, gs, ss, zsem):
    cid = lax.axis_index("c")
    sid = lax.axis_index("s")
    wid = cid * NS + sid
    base = wid * PER_TILE

    def _issue_gather(step, bi):
        off = base + step * B
        pltpu.async_copy(x_hbm.at[pl.ds(off, B)], xbs[bi], gs[bi])
        pltpu.async_copy(seg_hbm.at[pl.ds(off, B)], ibs[bi], gs[bi])

    def _wait_gather(step, bi):
        off = base + step * B
        pltpu.make_async_copy(x_hbm.at[pl.ds(off, B)], xbs[bi], gs[bi]).wait()
        pltpu.make_async_copy(seg_hbm.at[pl.ds(off, B)], ibs[bi], gs[bi]).wait()

    def _issue_scatter(bi):
        pltpu.async_copy(xbs[bi], acc.at[ibs[bi]], ss[bi], add=True)

    def _wait_scatter(bi):
        pltpu.make_async_copy(xbs[bi], acc.at[ibs[bi]], ss[bi]).wait()

    one16 = jnp.ones((16,), jnp.float32)

    def _hist(bi):
        ib = ibs[bi]
        for j in range(B // 16):
            idx = ib[pl.ds(j * 16, 16)]
            plsc.addupdate_scatter(hist, (idx,), one16)

    _issue_gather(0, 0)
    _issue_gather(1, 1)

    zero16 = jnp.zeros((16,), jnp.float32)
    zb = xbs[2]

    def _zrow(i, _):
        def _zcol(j, _):
            zb[i, pl.ds(j * 16, 16)] = zero16
            return 0
        return lax.fori_loop(0, D // 16, _zcol, 0)

    lax.fori_loop(0, B, _zrow, 0)

    rbase = sid * SEG_PER_TILE
    for k in range(SEG_PER_TILE // B):
        pltpu.async_copy(zb, acc.at[pl.ds(rbase + k * B, B)], zsem)

    def _zhist(i, _):
        hist[pl.ds(i * 16, 16)] = zero16
        return 0

    lax.fori_loop(0, SEG_PAD // 16, _zhist, 0)

    for k in range(SEG_PER_TILE // B):
        pltpu.make_async_copy(zb, acc.at[pl.ds(rbase + k * B, B)], zsem).wait()
    plsc.subcore_barrier()

    _wait_gather(0, 0)
    _hist(0)
    _issue_scatter(0)
    _issue_gather(2, 2)
    _wait_gather(1, 1)
    _hist(1)
    _issue_scatter(1)

    def _position(s, jb, jn):
        _wait_scatter(jn)
        _issue_gather(s + 1, jn)
        _wait_gather(s, jb)
        _hist(jb)
        _issue_scatter(jb)

    n_macro = (STEPS - 3 - 2) // NBUF

    def _body(i, _):
        s0 = 2 + NBUF * i
        for j in range(NBUF):
            _position(s0 + j, (2 + j) % NBUF, j % NBUF)
        return 0

    lax.fori_loop(0, n_macro, _body, 0)

    for s in range(2 + NBUF * n_macro, STEPS):
        jb = s % NBUF
        jn = (s + 1) % NBUF
        _wait_scatter(jn)
        if s + 1 < STEPS:
            _issue_gather(s + 1, jn)
        _wait_gather(s, jb)
        _hist(jb)
        _issue_scatter(jb)
    _wait_scatter((STEPS - 2) % NBUF)
    _wait_scatter((STEPS - 1) % NBUF)

    pltpu.sync_copy(hist, pcnt_hbm.at[wid])
    plsc.subcore_barrier()
    pltpu.sync_copy(acc.at[pl.ds(rbase, SEG_PER_TILE)],
                    psum_hbm.at[cid, pl.ds(rbase, SEG_PER_TILE)])


_RB = 1000


def _combine_body(ps_ref, pc_ref, o_ref):
    s = ps_ref[0] + ps_ref[1]
    c = jnp.sum(pc_ref[...], axis=1)
    o_ref[...] = s / jnp.maximum(c, 1.0)[:, None]


_combine = pl.pallas_call(
    _combine_body,
    grid=(N_SEG // _RB,),
    in_specs=[
        pl.BlockSpec((NC, _RB, D), lambda i: (0, i, 0)),
        pl.BlockSpec((_RB, NW), lambda i: (i, 0)),
    ],
    out_specs=pl.BlockSpec((_RB, D), lambda i: (i, 0)),
    out_shape=jax.ShapeDtypeStruct((N_SEG, D), jnp.float32),
)


def kernel(x, hyperedge_index, data, name):
    seg = hyperedge_index[1]
    psum, pcnt = _scatter_partials(x, seg)
    return _combine(psum, pcnt.T)


# speedup vs baseline: 11.3229x; 1.1072x over previous
import functools

import jax
import jax.numpy as jnp
from jax import lax
from jax.experimental import pallas as pl
from jax.experimental.pallas import tpu as pltpu
from jax.experimental.pallas import tpu_sc as plsc

N_SEG = 10000
SEG_PAD = 10240
N_INC = 320000
D = 128
NC = 2
NS = 16
DH = D // NC
PER_TILE = N_INC // NS
B = 128
FSTEPS = PER_TILE // B
TAIL = PER_TILE - FSTEPS * B
NBUF = 4
SEG_PER_TILE = SEG_PAD // NS
CH = 80
NCH = SEG_PER_TILE // CH

_mesh = plsc.VectorSubcoreMesh(
    core_axis_name="c", subcore_axis_name="s", num_cores=NC, num_subcores=NS
)


@functools.partial(
    pl.kernel,
    out_type=jax.ShapeDtypeStruct((N_SEG, D), jnp.float32),
    mesh=_mesh,
    compiler_params=pltpu.CompilerParams(
        use_tc_tiling_on_sc=False, needs_layout_passes=False
    ),
    scratch_types=[
        pltpu.VMEM_SHARED((SEG_PAD, DH), jnp.float32),
        pltpu.VMEM_SHARED((NS, SEG_PAD), jnp.float32),
        [pltpu.VMEM((B, DH), jnp.float32)] * NBUF,
        [pltpu.VMEM((B,), jnp.int32)] * NBUF,
        pltpu.VMEM((SEG_PAD,), jnp.float32),
        pltpu.VMEM((NS, CH), jnp.float32),
        pltpu.VMEM((SEG_PER_TILE,), jnp.float32),
        [pltpu.VMEM((CH, DH), jnp.float32)] * 2,
        [pltpu.VMEM((CH, DH), jnp.float32)] * 2,
        [pltpu.SemaphoreType.DMA] * NBUF,
        [pltpu.SemaphoreType.DMA] * NBUF,
        pltpu.SemaphoreType.DMA,
        [pltpu.SemaphoreType.DMA] * 2,
    ],
)
def _scatter_mean(x_hbm, seg_hbm, out_hbm,
                  acc, hstage, xbs, ibs, hist, hbuf, invb, abufs, obufs,
                  gs, ss, zsem, osem):
    cid = lax.axis_index("c")
    sid = lax.axis_index("s")
    base = sid * PER_TILE
    col0 = cid * DH

    def _issue_gather(step, bi, n=B):
        off = base + step * B
        pltpu.async_copy(x_hbm.at[pl.ds(off, n), pl.ds(col0, DH)],
                         xbs[bi].at[pl.ds(0, n)], gs[bi])
        pltpu.async_copy(seg_hbm.at[pl.ds(off, n)],
                         ibs[bi].at[pl.ds(0, n)], gs[bi])

    def _wait_gather(step, bi, n=B):
        off = base + step * B
        pltpu.make_async_copy(x_hbm.at[pl.ds(off, n), pl.ds(col0, DH)],
                              xbs[bi].at[pl.ds(0, n)], gs[bi]).wait()
        pltpu.make_async_copy(seg_hbm.at[pl.ds(off, n)],
                              ibs[bi].at[pl.ds(0, n)], gs[bi]).wait()

    def _issue_scatter(bi, n=B):
        pltpu.async_copy(xbs[bi].at[pl.ds(0, n)],
                         acc.at[ibs[bi].at[pl.ds(0, n)]], ss[bi], add=True)

    def _wait_scatter(bi, n=B):
        pltpu.make_async_copy(xbs[bi].at[pl.ds(0, n)],
                              acc.at[ibs[bi].at[pl.ds(0, n)]], ss[bi]).wait()

    one16 = jnp.ones((16,), jnp.float32)

    def _hist(bi, n=B):
        ib = ibs[bi]
        for j in range(n // 16):
            idx = ib[pl.ds(j * 16, 16)]
            plsc.addupdate_scatter(hist, (idx,), one16)

    _issue_gather(0, 0)
    _issue_gather(1, 1)

    zero16 = jnp.zeros((16,), jnp.float32)
    zb = xbs[NBUF - 1]

    def _zrow(i, _):
        def _zcol(j, _):
            zb[i, pl.ds(j * 16, 16)] = zero16
            return 0
        return lax.fori_loop(0, DH // 16, _zcol, 0)

    lax.fori_loop(0, B, _zrow, 0)

    rbase_pad = sid * SEG_PER_TILE
    for k in range(SEG_PER_TILE // B):
        pltpu.async_copy(zb, acc.at[pl.ds(rbase_pad + k * B, B)], zsem)

    def _zhist(i, _):
        hist[pl.ds(i * 16, 16)] = zero16
        return 0

    lax.fori_loop(0, SEG_PAD // 16, _zhist, 0)

    for k in range(SEG_PER_TILE // B):
        pltpu.make_async_copy(zb, acc.at[pl.ds(rbase_pad + k * B, B)],
                              zsem).wait()
    plsc.subcore_barrier()

    _wait_gather(0, 0)
    _hist(0)
    _issue_scatter(0)
    _issue_gather(2, 2)
    _wait_gather(1, 1)
    _hist(1)
    _issue_scatter(1)
    _issue_gather(3, 3)

    n_macro = (FSTEPS - 2 - 2) // NBUF

    def _body(i, _):
        s0 = 2 + NBUF * i
        for j in range(NBUF):
            s = s0 + j
            jb = (2 + j) % NBUF
            jn = j % NBUF
            _wait_scatter(jn)
            _issue_gather(s + 2, jn)
            _wait_gather(s, jb)
            _hist(jb)
            _issue_scatter(jb)
        return 0

    lax.fori_loop(0, n_macro, _body, 0)

    for s in range(2 + NBUF * n_macro, FSTEPS + 1):
        n = B if s < FSTEPS else TAIL
        jb = s % NBUF
        jn = (s + 2) % NBUF
        if s + 2 <= FSTEPS:
            _wait_scatter(jn)
            _issue_gather(s + 2, jn, n=(B if s + 2 < FSTEPS else TAIL))
        _wait_gather(s, jb, n=n)
        _hist(jb, n=n)
        _issue_scatter(jb, n=n)
    for s in range(FSTEPS - 3, FSTEPS + 1):
        _wait_scatter(s % NBUF, n=(B if s < FSTEPS else TAIL))

    pltpu.sync_copy(hist, hstage.at[sid])
    plsc.subcore_barrier()

    rbase = jnp.minimum(sid * SEG_PER_TILE, N_SEG - SEG_PER_TILE)

    def _issue_in(k, slot):
        pltpu.async_copy(acc.at[pl.ds(rbase + k * CH, CH)], abufs[slot],
                         gs[slot])

    def _wait_in(k, slot):
        pltpu.make_async_copy(acc.at[pl.ds(rbase + k * CH, CH)], abufs[slot],
                              gs[slot]).wait()

    def _issue_out(k, slot):
        pltpu.async_copy(obufs[slot],
                         out_hbm.at[pl.ds(rbase + k * CH, CH),
                                    pl.ds(col0, DH)], osem[slot])

    def _wait_out(k, slot):
        pltpu.make_async_copy(obufs[slot],
                              out_hbm.at[pl.ds(rbase + k * CH, CH),
                                         pl.ds(col0, DH)], osem[slot]).wait()

    _issue_in(0, 0)
    _issue_in(1, 1)

    for k in range(NCH):
        pltpu.sync_copy(hstage.at[:, pl.ds(rbase + k * CH, CH)], hbuf)

        def _invg(g, _, k=k):
            sl = pl.ds(g * 16, 16)
            c = hbuf[0, sl]
            for t in range(1, NS):
                c = c + hbuf[t, sl]
            invb[pl.ds(k * CH + g * 16, 16)] = 1.0 / jnp.maximum(c, 1.0)
            return 0

        lax.fori_loop(0, CH // 16, _invg, 0)

    def _compute(k, slot):
        ab, ob = abufs[slot], obufs[slot]

        def _rr(rr, _):
            cvec = invb[pl.ds(k * CH + rr * 16, 16)]
            for j in range(16):
                c = cvec[j]
                r = rr * 16 + j

                def _col(g, _, r=r, c=c):
                    sl = pl.ds(g * 16, 16)
                    ob[r, sl] = ab[r, sl] * c
                    return 0

                lax.fori_loop(0, DH // 16, _col, 0)
            return 0

        lax.fori_loop(0, CH // 16, _rr, 0)

    for k in range(NCH):
        slot = k % 2
        _wait_in(k, slot)
        if k >= 2:
            _wait_out(k - 2, slot)
        _compute(k, slot)
        if k + 2 < NCH:
            _issue_in(k + 2, slot)
        _issue_out(k, slot)
    _wait_out(NCH - 2, (NCH - 2) % 2)
    _wait_out(NCH - 1, (NCH - 1) % 2)


def kernel(x, hyperedge_index, data, name):
    seg = hyperedge_index[1]
    return _scatter_mean(x, seg)


# speedup vs baseline: 12.2361x; 1.0807x over previous
import functools

import jax
import jax.numpy as jnp
from jax import lax
from jax.experimental import pallas as pl
from jax.experimental.pallas import tpu as pltpu
from jax.experimental.pallas import tpu_sc as plsc

N_SEG = 10000
SEG_PAD = 10240
N_INC = 320000
D = 128
NC = 2
NS = 16
DH = D // NC
PER_TILE = N_INC // NS
B = 192
FSTEPS = PER_TILE // B
TAIL = PER_TILE - FSTEPS * B
NBUF = 3
SEG_PER_TILE = SEG_PAD // NS
CH = 80
NCH = SEG_PER_TILE // CH

_mesh = plsc.VectorSubcoreMesh(
    core_axis_name="c", subcore_axis_name="s", num_cores=NC, num_subcores=NS
)


@functools.partial(
    pl.kernel,
    out_type=jax.ShapeDtypeStruct((N_SEG, D), jnp.float32),
    mesh=_mesh,
    compiler_params=pltpu.CompilerParams(
        use_tc_tiling_on_sc=False, needs_layout_passes=False
    ),
    scratch_types=[
        pltpu.VMEM_SHARED((SEG_PAD, DH), jnp.float32),
        pltpu.VMEM_SHARED((NS, SEG_PAD), jnp.float32),
        [pltpu.VMEM((B, DH), jnp.float32)] * NBUF,
        [pltpu.VMEM((B,), jnp.int32)] * NBUF,
        pltpu.VMEM((SEG_PAD,), jnp.float32),
        pltpu.VMEM((NS, CH), jnp.float32),
        pltpu.VMEM((SEG_PER_TILE,), jnp.float32),
        [pltpu.VMEM((CH, DH), jnp.float32)] * 2,
        [pltpu.VMEM((CH, DH), jnp.float32)] * 2,
        [pltpu.SemaphoreType.DMA] * NBUF,
        [pltpu.SemaphoreType.DMA] * NBUF,
        pltpu.SemaphoreType.DMA,
        [pltpu.SemaphoreType.DMA] * 2,
    ],
)
def _scatter_mean(x_hbm, hei_hbm, out_hbm,
                  acc, hstage, xbs, ibs, hist, hbuf, invb, abufs, obufs,
                  gs, ss, zsem, osem):
    cid = lax.axis_index("c")
    sid = lax.axis_index("s")
    base = sid * PER_TILE
    col0 = cid * DH

    def _issue_gather(step, bi, n=B):
        off = base + step * B
        pltpu.async_copy(x_hbm.at[pl.ds(off, n), pl.ds(col0, DH)],
                         xbs[bi].at[pl.ds(0, n)], gs[bi])
        pltpu.async_copy(hei_hbm.at[1, pl.ds(off, n)],
                         ibs[bi].at[pl.ds(0, n)], gs[bi])

    def _wait_gather(step, bi, n=B):
        off = base + step * B
        pltpu.make_async_copy(x_hbm.at[pl.ds(off, n), pl.ds(col0, DH)],
                              xbs[bi].at[pl.ds(0, n)], gs[bi]).wait()
        pltpu.make_async_copy(hei_hbm.at[1, pl.ds(off, n)],
                              ibs[bi].at[pl.ds(0, n)], gs[bi]).wait()

    def _issue_scatter(bi, n=B):
        for o in range(0, n, 128):
            w = min(128, n - o)
            pltpu.async_copy(xbs[bi].at[pl.ds(o, w)],
                             acc.at[ibs[bi].at[pl.ds(o, w)]], ss[bi], add=True)

    def _wait_scatter(bi, n=B):
        for o in range(0, n, 128):
            w = min(128, n - o)
            pltpu.make_async_copy(xbs[bi].at[pl.ds(o, w)],
                                  acc.at[ibs[bi].at[pl.ds(o, w)]],
                                  ss[bi]).wait()

    one16 = jnp.ones((16,), jnp.float32)

    def _hist(bi, n=B):
        ib = ibs[bi]
        for j in range(n // 16):
            idx = ib[pl.ds(j * 16, 16)]
            plsc.addupdate_scatter(hist, (idx,), one16)

    _issue_gather(0, 0)
    _issue_gather(1, 1)

    zero16 = jnp.zeros((16,), jnp.float32)
    zb = xbs[NBUF - 1]

    def _zrow(i, _):
        def _zcol(j, _):
            zb[i, pl.ds(j * 16, 16)] = zero16
            return 0
        return lax.fori_loop(0, DH // 16, _zcol, 0)

    lax.fori_loop(0, B, _zrow, 0)

    rbase_pad = sid * SEG_PER_TILE
    for k in range(SEG_PER_TILE // 128):
        pltpu.async_copy(zb.at[pl.ds(0, 128)],
                         acc.at[pl.ds(rbase_pad + k * 128, 128)], zsem)

    def _zhist(i, _):
        hist[pl.ds(i * 16, 16)] = zero16
        return 0

    lax.fori_loop(0, SEG_PAD // 16, _zhist, 0)

    for k in range(SEG_PER_TILE // 128):
        pltpu.make_async_copy(zb.at[pl.ds(0, 128)],
                              acc.at[pl.ds(rbase_pad + k * 128, 128)],
                              zsem).wait()
    plsc.subcore_barrier()

    _wait_gather(0, 0)
    _issue_scatter(0)
    _hist(0)
    _issue_gather(2, 2)
    _wait_gather(1, 1)
    _issue_scatter(1)
    _hist(1)

    n_macro = (FSTEPS + 1 - 2 - 2) // NBUF

    def _body(i, _):
        s0 = 2 + NBUF * i
        for j in range(NBUF):
            s = s0 + j
            jb = (2 + j) % NBUF
            jn = j % NBUF
            _wait_scatter(jn)
            _issue_gather(s + 1, jn)
            _wait_gather(s, jb)
            _issue_scatter(jb)
            _hist(jb)
        return 0

    lax.fori_loop(0, n_macro, _body, 0)

    for s in range(2 + NBUF * n_macro, FSTEPS + 1):
        n = B if s < FSTEPS else TAIL
        jb = s % NBUF
        jn = (s + 1) % NBUF
        _wait_scatter(jn)
        if s + 1 <= FSTEPS:
            _issue_gather(s + 1, jn, n=(B if s + 1 < FSTEPS else TAIL))
        _wait_gather(s, jb, n=n)
        _issue_scatter(jb, n=n)
        _hist(jb, n=n)
    _wait_scatter((FSTEPS - 1) % NBUF, n=B)
    _wait_scatter(FSTEPS % NBUF, n=TAIL)

    pltpu.sync_copy(hist, hstage.at[sid])
    plsc.subcore_barrier()

    rbase = jnp.minimum(sid * SEG_PER_TILE, N_SEG - SEG_PER_TILE)

    def _issue_in(k, slot):
        pltpu.async_copy(acc.at[pl.ds(rbase + k * CH, CH)], abufs[slot],
                         gs[slot])

    def _wait_in(k, slot):
        pltpu.make_async_copy(acc.at[pl.ds(rbase + k * CH, CH)], abufs[slot],
                              gs[slot]).wait()

    def _issue_out(k, slot):
        pltpu.async_copy(obufs[slot],
                         out_hbm.at[pl.ds(rbase + k * CH, CH),
                                    pl.ds(col0, DH)], osem[slot])

    def _wait_out(k, slot):
        pltpu.make_async_copy(obufs[slot],
                              out_hbm.at[pl.ds(rbase + k * CH, CH),
                                         pl.ds(col0, DH)], osem[slot]).wait()

    _issue_in(0, 0)
    _issue_in(1, 1)

    for k in range(NCH):
        pltpu.sync_copy(hstage.at[:, pl.ds(rbase + k * CH, CH)], hbuf)

        def _invg(g, _, k=k):
            sl = pl.ds(g * 16, 16)
            c = hbuf[0, sl]
            for t in range(1, NS):
                c = c + hbuf[t, sl]
            invb[pl.ds(k * CH + g * 16, 16)] = 1.0 / jnp.maximum(c, 1.0)
            return 0

        lax.fori_loop(0, CH // 16, _invg, 0)

    def _compute(k, slot):
        ab, ob = abufs[slot], obufs[slot]

        def _rr(rr, _):
            cvec = invb[pl.ds(k * CH + rr * 16, 16)]
            for j in range(16):
                c = cvec[j]
                r = rr * 16 + j
                for g in range(DH // 16):
                    sl = pl.ds(g * 16, 16)
                    ob[r, sl] = ab[r, sl] * c
            return 0

        lax.fori_loop(0, CH // 16, _rr, 0)

    for k in range(NCH):
        slot = k % 2
        _wait_in(k, slot)
        if k >= 2:
            _wait_out(k - 2, slot)
        _compute(k, slot)
        if k + 2 < NCH:
            _issue_in(k + 2, slot)
        _issue_out(k, slot)
    _wait_out(NCH - 2, (NCH - 2) % 2)
    _wait_out(NCH - 1, (NCH - 1) % 2)


def kernel(x, hyperedge_index, data, name):
    return _scatter_mean(x, hyperedge_index)
